# baseline (speedup 1.0000x reference)
import functools

import jax
import jax.numpy as jnp
from jax import lax
from jax.experimental import pallas as pl
from jax.experimental.pallas import tpu as pltpu
from jax.experimental.pallas import tpu_sc as plsc

N = 100000
E = 1600000
NP = 100352
NC = 2
NS = 16
NW = NC * NS
RPT = NP // NS
ZR = 3136
C = 2000
C1 = 800
ZR1 = 784
EPW = E // NW
EPT = E // NS
BR = 7168
GRID = NP // BR

_mesh = plsc.VectorSubcoreMesh(core_axis_name="c", subcore_axis_name="s")
_sc_params = pltpu.CompilerParams(use_tc_tiling_on_sc=False,
                                  needs_layout_passes=False)


def _fill(ref, n, value):
    @pl.loop(0, n // 16)
    def _(j):
        ref[pl.ds(j * 16, 16)] = jnp.full((16,), value, jnp.float32)


@functools.partial(
    pl.kernel,
    out_type=jax.ShapeDtypeStruct((NC * NP,), jnp.float32),
    mesh=_mesh,
    scratch_types=[
        pltpu.VMEM((2, C), jnp.int32),
        pltpu.VMEM((C,), jnp.float32),
        pltpu.VMEM((ZR,), jnp.float32),
        pltpu.SemaphoreType.DMA,
        pltpu.VMEM_SHARED((NP,), jnp.float32),
    ],
    compiler_params=_sc_params,
)
def _sc_deg(ei_hbm, out_hbm, dst_v, ones_v, z_v, isem, acc):
    cid = lax.axis_index("c")
    sid = lax.axis_index("s")
    wid = sid * NC + cid
    nchunk = EPW // C
    _fill(ones_v, C, 1.0)
    _fill(z_v, ZR, 0.0)

    @pl.loop(0, RPT // ZR)
    def _(j):
        pltpu.sync_copy(z_v, acc.at[pl.ds(sid * RPT + j * ZR, ZR)])

    plsc.subcore_barrier()

    def _idx(k, b):
        off = wid * EPW + k * C
        return pltpu.async_copy(ei_hbm.at[1, pl.ds(off, C)],
                                dst_v.at[b], isem)

    _idx(0, 0)

    @pl.loop(0, nchunk)
    def _(k):
        b = lax.rem(k, 2)
        pltpu.make_async_copy(ei_hbm.at[1, pl.ds(0, C)],
                              dst_v.at[b], isem).wait()

        @pl.when(k + 1 < nchunk)
        def _():
            _idx(k + 1, 1 - b)

        pltpu.sync_copy(ones_v, acc.at[dst_v.at[b]], add=True)

    plsc.subcore_barrier()
    pltpu.sync_copy(acc.at[pl.ds(sid * RPT, RPT)],
                    out_hbm.at[pl.ds(cid * NP + sid * RPT, RPT)])


@functools.partial(
    pl.kernel,
    out_type=jax.ShapeDtypeStruct((NC * NP, 16), jnp.float32),
    mesh=_mesh,
    scratch_types=[
        pltpu.VMEM((2, C1), jnp.int32),
        pltpu.VMEM((2, C1), jnp.int32),
        pltpu.VMEM((2, C1, 16), jnp.float32),
        pltpu.SemaphoreType.DMA,
        pltpu.SemaphoreType.DMA,
        pltpu.VMEM_SHARED((NP, 16), jnp.float32),
    ],
    compiler_params=_sc_params,
)
def _sc_agg30(ei_hbm, ta_hbm, tb_hbm, out_hbm,
              src_v, dst_v, rows_v, gsem, isem, acc):
    cid = lax.axis_index("c")
    sid = lax.axis_index("s")
    nself = (RPT + C1 - 1) // C1
    nchunk = EPT // C1 + nself

    @pl.loop(0, ZR1)
    def _(j):
        rows_v[0, j, :] = jnp.zeros((16,), jnp.float32)

    @pl.loop(0, RPT // ZR1)
    def _(j):
        pltpu.sync_copy(rows_v.at[0, pl.ds(0, ZR1), :],
                        acc.at[pl.ds(sid * RPT + j * ZR1, ZR1), :])

    plsc.subcore_barrier()

    nedge = EPT // C1

    def _buf_prep(k, b):
        @pl.when(k < nedge)
        def _():
            off = sid * EPT + k * C1
            pltpu.async_copy(ei_hbm.at[0, pl.ds(off, C1)], src_v.at[b], isem)
            pltpu.async_copy(ei_hbm.at[1, pl.ds(off, C1)], dst_v.at[b], isem)
            pltpu.make_async_copy(ei_hbm.at[0, pl.ds(0, C1)],
                                  src_v.at[b], isem).wait()
            pltpu.make_async_copy(ei_hbm.at[1, pl.ds(0, C1)],
                                  dst_v.at[b], isem).wait()

        @pl.when(k >= nedge)
        def _():
            base = sid * RPT + (k - nedge) * C1
            lim = (sid + 1) * RPT

            @pl.loop(0, C1 // 16)
            def _(t):
                v = base + t * 16 + lax.iota(jnp.int32, 16)
                v = jnp.where(v < lim, v, NP - 1)
                src_v[b, pl.ds(t * 16, 16)] = v
                dst_v[b, pl.ds(t * 16, 16)] = v

    H = C1 // 2

    def _gather_start(k, b):
        @pl.when(cid == 0)
        def _():
            pltpu.async_copy(ta_hbm.at[src_v.at[b, pl.ds(0, H)]],
                             rows_v.at[b, pl.ds(0, H), :], gsem)
            pltpu.async_copy(ta_hbm.at[src_v.at[b, pl.ds(H, H)]],
                             rows_v.at[b, pl.ds(H, H), :], gsem)

        @pl.when(cid == 1)
        def _():
            pltpu.async_copy(tb_hbm.at[src_v.at[b, pl.ds(0, H)]],
                             rows_v.at[b, pl.ds(0, H), :], gsem)
            pltpu.async_copy(tb_hbm.at[src_v.at[b, pl.ds(H, H)]],
                             rows_v.at[b, pl.ds(H, H), :], gsem)

    def _gather_wait(b):
        pltpu.make_async_copy(ta_hbm.at[src_v.at[b]], rows_v.at[b],
                              gsem).wait()

    _buf_prep(0, 0)
    _gather_start(0, 0)
    _buf_prep(1, 1)

    @pl.loop(0, nchunk)
    def _(k):
        b = lax.rem(k, 2)
        nb = 1 - b
        _gather_wait(b)

        @pl.when(k + 1 < nchunk)
        def _():
            _gather_start(k + 1, nb)

        pltpu.sync_copy(rows_v.at[b], acc.at[dst_v.at[b]], add=True)

        @pl.when(k + 2 < nchunk)
        def _():
            _buf_prep(k + 2, b)

    plsc.subcore_barrier()
    pltpu.sync_copy(acc.at[pl.ds(sid * RPT, RPT), :],
                    out_hbm.at[pl.ds(cid * NP + sid * RPT, RPT), :])


@functools.partial(
    pl.kernel,
    out_type=jax.ShapeDtypeStruct((NC * NP,), jnp.float32),
    mesh=_mesh,
    scratch_types=[
        pltpu.VMEM((2, C), jnp.int32),
        pltpu.VMEM((2, C), jnp.int32),
        pltpu.VMEM((C,), jnp.float32),
        pltpu.VMEM((NP,), jnp.float32),
        pltpu.VMEM((ZR,), jnp.float32),
        pltpu.SemaphoreType.DMA,
        pltpu.VMEM_SHARED((NP,), jnp.float32),
    ],
    compiler_params=_sc_params,
)
def _sc_agg1(ei_hbm, st_hbm, out_hbm,
             src_v, dst_v, vals_v, table_v, z_v, isem, acc):
    cid = lax.axis_index("c")
    sid = lax.axis_index("s")
    wid = sid * NC + cid
    nchunk = EPW // C
    pltpu.sync_copy(st_hbm, table_v)
    _fill(z_v, ZR, 0.0)

    @pl.loop(0, RPT // ZR)
    def _(j):
        pltpu.sync_copy(z_v, acc.at[pl.ds(sid * RPT + j * ZR, ZR)])

    plsc.subcore_barrier()

    def _idx(k, b):
        off = wid * EPW + k * C
        pltpu.async_copy(ei_hbm.at[0, pl.ds(off, C)], src_v.at[b], isem)
        pltpu.async_copy(ei_hbm.at[1, pl.ds(off, C)], dst_v.at[b], isem)

    def _idx_wait(b):
        pltpu.make_async_copy(ei_hbm.at[0, pl.ds(0, C)],
                              src_v.at[b], isem).wait()
        pltpu.make_async_copy(ei_hbm.at[1, pl.ds(0, C)],
                              dst_v.at[b], isem).wait()

    _idx(0, 0)

    @pl.loop(0, nchunk)
    def _(k):
        b = lax.rem(k, 2)
        _idx_wait(b)

        @pl.when(k + 1 < nchunk)
        def _():
            _idx(k + 1, 1 - b)

        @pl.loop(0, C // 16)
        def _(j):
            idx = src_v[b, pl.ds(j * 16, 16)]
            vals_v[pl.ds(j * 16, 16)] = plsc.load_gather(table_v, [idx])

        pltpu.sync_copy(vals_v, acc.at[dst_v.at[b]], add=True)

    plsc.subcore_barrier()
    pltpu.sync_copy(acc.at[pl.ds(sid * RPT, RPT)],
                    out_hbm.at[pl.ds(cid * NP + sid * RPT, RPT)])


def _tc1_body(x_ref, w1_ref, p0_ref, p1_ref, ta_ref, tb_ref, d_ref):
    deg = p0_ref[...] + p1_ref[...] + 1.0
    dis = lax.rsqrt(deg)
    h = jnp.dot(x_ref[...], w1_ref[...], preferred_element_type=jnp.float32)
    ht = h * dis[:, None]
    ta_ref[...] = ht[:, :16]
    tb_ref[...] = ht[:, 16:]
    d_ref[...] = dis


def _tc2_body(aa_ref, ab_ref, d_ref, b1a_ref,
              b1b_ref, w2a_ref, w2b_ref, st_ref):
    dd = d_ref[...][:, None]
    h1a = jax.nn.relu(dd * aa_ref[...] + b1a_ref[...])
    h1b = jax.nn.relu(dd * ab_ref[...] + b1b_ref[...])
    s = (jnp.dot(h1a, w2a_ref[...], preferred_element_type=jnp.float32)
         + jnp.dot(h1b, w2b_ref[...], preferred_element_type=jnp.float32))
    st_ref[...] = s[:, 0] * d_ref[...]


def _tc3_body(q0_ref, q1_ref, st_ref, d_ref, b2_ref, o_ref):
    z = d_ref[...] * (q0_ref[...] + q1_ref[...] + st_ref[...]) + b2_ref[0]
    o_ref[...] = jax.nn.sigmoid(z)


def _rows(width):
    return pl.BlockSpec((BR, width), lambda i: (i, 0))


def _rows2(width):
    return pl.BlockSpec((BR, width), lambda i: (i + GRID, 0))


def _flat():
    return pl.BlockSpec((BR,), lambda i: (i,))


def _flat2():
    return pl.BlockSpec((BR,), lambda i: (i + GRID,))


def _whole(shape):
    return pl.BlockSpec(shape, lambda i: (0,) * len(shape))


_tc1 = pl.pallas_call(
    _tc1_body,
    grid=(GRID,),
    in_specs=[_rows(32), _whole((32, 32)), _flat(), _flat2()],
    out_specs=[_rows(16), _rows(16), _flat()],
    out_shape=[
        jax.ShapeDtypeStruct((NP, 16), jnp.float32),
        jax.ShapeDtypeStruct((NP, 16), jnp.float32),
        jax.ShapeDtypeStruct((NP,), jnp.float32),
    ],
)

_tc2 = pl.pallas_call(
    _tc2_body,
    grid=(GRID,),
    in_specs=[_rows(16), _rows2(16),
              _flat(), _whole((1, 16)), _whole((1, 16)),
              _whole((16, 1)), _whole((16, 1))],
    out_specs=_flat(),
    out_shape=jax.ShapeDtypeStruct((NP,), jnp.float32),
)

_tc3 = pl.pallas_call(
    _tc3_body,
    grid=(GRID,),
    in_specs=[_flat(), _flat2(), _flat(), _flat(), _whole((1,))],
    out_specs=_flat(),
    out_shape=jax.ShapeDtypeStruct((NP,), jnp.float32),
)


def kernel(x, edge_index, W1, b1, W2, b2):
    xp = jnp.pad(x, ((0, NP - N), (0, 0)))
    w1p = jnp.pad(W1, ((0, 0), (0, 2)))
    b1p = jnp.pad(b1, (0, 2)).reshape(1, 32)
    w2p = jnp.pad(W2[:, 0], (0, 2)).reshape(1, 32)

    degp = _sc_deg(edge_index)

    ta, tb, d = _tc1(xp, w1p, degp, degp)

    accf = _sc_agg30(edge_index, ta, tb)

    st = _tc2(accf, accf, d, b1p[:, :16], b1p[:, 16:],
              w2p[0, :16].reshape(16, 1), w2p[0, 16:].reshape(16, 1))

    acc2 = _sc_agg1(edge_index, st)

    out = _tc3(acc2, acc2, st, d, b2)
    return out[:N, None]


# speedup vs baseline: 1.0185x; 1.0185x over previous
import functools

import jax
import jax.numpy as jnp
from jax import lax
from jax.experimental import pallas as pl
from jax.experimental.pallas import tpu as pltpu
from jax.experimental.pallas import tpu_sc as plsc

N = 100000
E = 1600000
NP = 100352
NC = 2
NS = 16
NW = NC * NS
RPT = NP // NS
ZR = 3136
C = 2000
C1 = 800
ZR1 = 784
EPW = E // NW
EPT = E // NS
BR = 7168
GRID = NP // BR

_mesh = plsc.VectorSubcoreMesh(core_axis_name="c", subcore_axis_name="s")
_sc_params = pltpu.CompilerParams(use_tc_tiling_on_sc=False,
                                  needs_layout_passes=False)


def _fill(ref, n, value):
    @pl.loop(0, n // 16)
    def _(j):
        ref[pl.ds(j * 16, 16)] = jnp.full((16,), value, jnp.float32)


@functools.partial(
    pl.kernel,
    out_type=jax.ShapeDtypeStruct((NC * NP,), jnp.float32),
    mesh=_mesh,
    scratch_types=[
        pltpu.VMEM((2, C), jnp.int32),
        pltpu.VMEM((C,), jnp.float32),
        pltpu.VMEM((ZR,), jnp.float32),
        pltpu.SemaphoreType.DMA,
        pltpu.VMEM_SHARED((NP,), jnp.float32),
    ],
    compiler_params=_sc_params,
)
def _sc_deg(ei_hbm, out_hbm, dst_v, ones_v, z_v, isem, acc):
    cid = lax.axis_index("c")
    sid = lax.axis_index("s")
    wid = sid * NC + cid
    nchunk = EPW // C
    _fill(ones_v, C, 1.0)
    _fill(z_v, ZR, 0.0)

    @pl.loop(0, RPT // ZR)
    def _(j):
        pltpu.sync_copy(z_v, acc.at[pl.ds(sid * RPT + j * ZR, ZR)])

    plsc.subcore_barrier()

    def _idx(k, b):
        off = wid * EPW + k * C
        return pltpu.async_copy(ei_hbm.at[1, pl.ds(off, C)],
                                dst_v.at[b], isem)

    _idx(0, 0)

    @pl.loop(0, nchunk)
    def _(k):
        b = lax.rem(k, 2)
        pltpu.make_async_copy(ei_hbm.at[1, pl.ds(0, C)],
                              dst_v.at[b], isem).wait()

        @pl.when(k + 1 < nchunk)
        def _():
            _idx(k + 1, 1 - b)

        pltpu.sync_copy(ones_v, acc.at[dst_v.at[b]], add=True)

    plsc.subcore_barrier()
    pltpu.sync_copy(acc.at[pl.ds(sid * RPT, RPT)],
                    out_hbm.at[pl.ds(cid * NP + sid * RPT, RPT)])


@functools.partial(
    pl.kernel,
    out_type=jax.ShapeDtypeStruct((NC * NP, 16), jnp.float32),
    mesh=_mesh,
    scratch_types=[
        pltpu.VMEM((2, C1), jnp.int32),
        pltpu.VMEM((2, C1), jnp.int32),
        pltpu.VMEM((2, C1, 16), jnp.float32),
        pltpu.SemaphoreType.DMA,
        pltpu.SemaphoreType.DMA,
        pltpu.VMEM_SHARED((NP, 16), jnp.float32),
    ],
    compiler_params=_sc_params,
)
def _sc_agg30(ei_hbm, ta_hbm, tb_hbm, out_hbm,
              src_v, dst_v, rows_v, gsem, isem, acc):
    cid = lax.axis_index("c")
    sid = lax.axis_index("s")
    nchunk = EPT // C1

    @pl.loop(0, ZR1)
    def _(j):
        rows_v[0, j, :] = jnp.zeros((16,), jnp.float32)

    @pl.loop(0, RPT // ZR1)
    def _(j):
        pltpu.sync_copy(rows_v.at[0, pl.ds(0, ZR1), :],
                        acc.at[pl.ds(sid * RPT + j * ZR1, ZR1), :])

    plsc.subcore_barrier()

    nedge = EPT // C1

    def _buf_prep(k, b):
        off = sid * EPT + k * C1
        pltpu.async_copy(ei_hbm.at[0, pl.ds(off, C1)], src_v.at[b], isem)
        pltpu.async_copy(ei_hbm.at[1, pl.ds(off, C1)], dst_v.at[b], isem)
        pltpu.make_async_copy(ei_hbm.at[0, pl.ds(0, C1)],
                              src_v.at[b], isem).wait()
        pltpu.make_async_copy(ei_hbm.at[1, pl.ds(0, C1)],
                              dst_v.at[b], isem).wait()

    H = C1 // 2

    def _gather_start(k, b):
        @pl.when(cid == 0)
        def _():
            pltpu.async_copy(ta_hbm.at[src_v.at[b, pl.ds(0, H)]],
                             rows_v.at[b, pl.ds(0, H), :], gsem)
            pltpu.async_copy(ta_hbm.at[src_v.at[b, pl.ds(H, H)]],
                             rows_v.at[b, pl.ds(H, H), :], gsem)

        @pl.when(cid == 1)
        def _():
            pltpu.async_copy(tb_hbm.at[src_v.at[b, pl.ds(0, H)]],
                             rows_v.at[b, pl.ds(0, H), :], gsem)
            pltpu.async_copy(tb_hbm.at[src_v.at[b, pl.ds(H, H)]],
                             rows_v.at[b, pl.ds(H, H), :], gsem)

    def _gather_wait(b):
        pltpu.make_async_copy(ta_hbm.at[src_v.at[b]], rows_v.at[b],
                              gsem).wait()

    _buf_prep(0, 0)
    _gather_start(0, 0)
    _buf_prep(1, 1)

    @pl.loop(0, nchunk)
    def _(k):
        b = lax.rem(k, 2)
        nb = 1 - b
        _gather_wait(b)

        @pl.when(k + 1 < nchunk)
        def _():
            _gather_start(k + 1, nb)

        pltpu.sync_copy(rows_v.at[b], acc.at[dst_v.at[b]], add=True)

        @pl.when(k + 2 < nchunk)
        def _():
            _buf_prep(k + 2, b)

    plsc.subcore_barrier()
    pltpu.sync_copy(acc.at[pl.ds(sid * RPT, RPT), :],
                    out_hbm.at[pl.ds(cid * NP + sid * RPT, RPT), :])


@functools.partial(
    pl.kernel,
    out_type=jax.ShapeDtypeStruct((NC * NP,), jnp.float32),
    mesh=_mesh,
    scratch_types=[
        pltpu.VMEM((2, C), jnp.int32),
        pltpu.VMEM((2, C), jnp.int32),
        pltpu.VMEM((C,), jnp.float32),
        pltpu.VMEM((NP,), jnp.float32),
        pltpu.VMEM((ZR,), jnp.float32),
        pltpu.SemaphoreType.DMA,
        pltpu.VMEM_SHARED((NP,), jnp.float32),
    ],
    compiler_params=_sc_params,
)
def _sc_agg1(ei_hbm, st_hbm, out_hbm,
             src_v, dst_v, vals_v, table_v, z_v, isem, acc):
    cid = lax.axis_index("c")
    sid = lax.axis_index("s")
    wid = sid * NC + cid
    nchunk = EPW // C
    pltpu.sync_copy(st_hbm, table_v)
    _fill(z_v, ZR, 0.0)

    @pl.loop(0, RPT // ZR)
    def _(j):
        pltpu.sync_copy(z_v, acc.at[pl.ds(sid * RPT + j * ZR, ZR)])

    plsc.subcore_barrier()

    def _idx(k, b):
        off = wid * EPW + k * C
        pltpu.async_copy(ei_hbm.at[0, pl.ds(off, C)], src_v.at[b], isem)
        pltpu.async_copy(ei_hbm.at[1, pl.ds(off, C)], dst_v.at[b], isem)

    def _idx_wait(b):
        pltpu.make_async_copy(ei_hbm.at[0, pl.ds(0, C)],
                              src_v.at[b], isem).wait()
        pltpu.make_async_copy(ei_hbm.at[1, pl.ds(0, C)],
                              dst_v.at[b], isem).wait()

    _idx(0, 0)

    @pl.loop(0, nchunk)
    def _(k):
        b = lax.rem(k, 2)
        _idx_wait(b)

        @pl.when(k + 1 < nchunk)
        def _():
            _idx(k + 1, 1 - b)

        @pl.loop(0, C // 16)
        def _(j):
            idx = src_v[b, pl.ds(j * 16, 16)]
            vals_v[pl.ds(j * 16, 16)] = plsc.load_gather(table_v, [idx])

        pltpu.sync_copy(vals_v, acc.at[dst_v.at[b]], add=True)

    plsc.subcore_barrier()
    pltpu.sync_copy(acc.at[pl.ds(sid * RPT, RPT)],
                    out_hbm.at[pl.ds(cid * NP + sid * RPT, RPT)])


def _tc1_body(x_ref, w1_ref, p0_ref, p1_ref, ta_ref, tb_ref, d_ref):
    deg = p0_ref[...] + p1_ref[...] + 1.0
    dis = lax.rsqrt(deg)
    h = jnp.dot(x_ref[...], w1_ref[...], preferred_element_type=jnp.float32)
    ht = h * dis[:, None]
    ta_ref[...] = ht[:, :16]
    tb_ref[...] = ht[:, 16:]
    d_ref[...] = dis


def _tc2_body(aa_ref, ab_ref, ta_ref, tb_ref, d_ref, b1a_ref,
              b1b_ref, w2a_ref, w2b_ref, st_ref):
    dd = d_ref[...][:, None]
    h1a = jax.nn.relu(dd * (aa_ref[...] + ta_ref[...]) + b1a_ref[...])
    h1b = jax.nn.relu(dd * (ab_ref[...] + tb_ref[...]) + b1b_ref[...])
    s = (jnp.dot(h1a, w2a_ref[...], preferred_element_type=jnp.float32)
         + jnp.dot(h1b, w2b_ref[...], preferred_element_type=jnp.float32))
    st_ref[...] = s[:, 0] * d_ref[...]


def _tc3_body(q0_ref, q1_ref, st_ref, d_ref, b2_ref, o_ref):
    z = d_ref[...] * (q0_ref[...] + q1_ref[...] + st_ref[...]) + b2_ref[0]
    o_ref[...] = jax.nn.sigmoid(z)


def _rows(width):
    return pl.BlockSpec((BR, width), lambda i: (i, 0))


def _rows2(width):
    return pl.BlockSpec((BR, width), lambda i: (i + GRID, 0))


def _flat():
    return pl.BlockSpec((BR,), lambda i: (i,))


def _flat2():
    return pl.BlockSpec((BR,), lambda i: (i + GRID,))


def _whole(shape):
    return pl.BlockSpec(shape, lambda i: (0,) * len(shape))


_tc1 = pl.pallas_call(
    _tc1_body,
    grid=(GRID,),
    in_specs=[_rows(32), _whole((32, 32)), _flat(), _flat2()],
    out_specs=[_rows(16), _rows(16), _flat()],
    out_shape=[
        jax.ShapeDtypeStruct((NP, 16), jnp.float32),
        jax.ShapeDtypeStruct((NP, 16), jnp.float32),
        jax.ShapeDtypeStruct((NP,), jnp.float32),
    ],
)

_tc2 = pl.pallas_call(
    _tc2_body,
    grid=(GRID,),
    in_specs=[_rows(16), _rows2(16), _rows(16), _rows(16),
              _flat(), _whole((1, 16)), _whole((1, 16)),
              _whole((16, 1)), _whole((16, 1))],
    out_specs=_flat(),
    out_shape=jax.ShapeDtypeStruct((NP,), jnp.float32),
)

_tc3 = pl.pallas_call(
    _tc3_body,
    grid=(GRID,),
    in_specs=[_flat(), _flat2(), _flat(), _flat(), _whole((1,))],
    out_specs=_flat(),
    out_shape=jax.ShapeDtypeStruct((NP,), jnp.float32),
)


def kernel(x, edge_index, W1, b1, W2, b2):
    xp = jnp.pad(x, ((0, NP - N), (0, 0)))
    w1p = jnp.pad(W1, ((0, 0), (0, 2)))
    b1p = jnp.pad(b1, (0, 2)).reshape(1, 32)
    w2p = jnp.pad(W2[:, 0], (0, 2)).reshape(1, 32)

    degp = _sc_deg(edge_index)

    ta, tb, d = _tc1(xp, w1p, degp, degp)

    accf = _sc_agg30(edge_index, ta, tb)

    st = _tc2(accf, accf, ta, tb, d, b1p[:, :16], b1p[:, 16:],
              w2p[0, :16].reshape(16, 1), w2p[0, 16:].reshape(16, 1))

    acc2 = _sc_agg1(edge_index, st)

    out = _tc3(acc2, acc2, st, d, b2)
    return out[:N, None]


# speedup vs baseline: 1.0376x; 1.0187x over previous
import functools

import jax
import jax.numpy as jnp
from jax import lax
from jax.experimental import pallas as pl
from jax.experimental.pallas import tpu as pltpu
from jax.experimental.pallas import tpu_sc as plsc

N = 100000
E = 1600000
NP = 100352
NC = 2
NS = 16
NW = NC * NS
RPT = NP // NS
ZR = 3136
C = 2000
C1 = 800
ZR1 = 784
EPW = E // NW
EPT = E // NS
BR = 7168
GRID = NP // BR

_mesh = plsc.VectorSubcoreMesh(core_axis_name="c", subcore_axis_name="s")
_sc_params = pltpu.CompilerParams(use_tc_tiling_on_sc=False,
                                  needs_layout_passes=False)


def _fill(ref, n, value):
    @pl.loop(0, n // 16)
    def _(j):
        ref[pl.ds(j * 16, 16)] = jnp.full((16,), value, jnp.float32)


@functools.partial(
    pl.kernel,
    out_type=jax.ShapeDtypeStruct((NC * NP,), jnp.float32),
    mesh=_mesh,
    scratch_types=[
        pltpu.VMEM((2, C), jnp.int32),
        pltpu.VMEM((C,), jnp.float32),
        pltpu.VMEM((ZR,), jnp.float32),
        pltpu.SemaphoreType.DMA,
        pltpu.VMEM_SHARED((NP,), jnp.float32),
    ],
    compiler_params=_sc_params,
)
def _sc_deg(ei_hbm, out_hbm, dst_v, ones_v, z_v, isem, acc):
    cid = lax.axis_index("c")
    sid = lax.axis_index("s")
    wid = sid * NC + cid
    nchunk = EPW // C
    _fill(ones_v, C, 1.0)
    _fill(z_v, ZR, 0.0)

    @pl.loop(0, RPT // ZR)
    def _(j):
        pltpu.sync_copy(z_v, acc.at[pl.ds(sid * RPT + j * ZR, ZR)])

    plsc.subcore_barrier()

    def _idx(k, b):
        off = wid * EPW + k * C
        return pltpu.async_copy(ei_hbm.at[1, pl.ds(off, C)],
                                dst_v.at[b], isem)

    _idx(0, 0)

    @pl.loop(0, nchunk)
    def _(k):
        b = lax.rem(k, 2)
        pltpu.make_async_copy(ei_hbm.at[1, pl.ds(0, C)],
                              dst_v.at[b], isem).wait()

        @pl.when(k + 1 < nchunk)
        def _():
            _idx(k + 1, 1 - b)

        pltpu.sync_copy(ones_v, acc.at[dst_v.at[b]], add=True)

    plsc.subcore_barrier()
    pltpu.sync_copy(acc.at[pl.ds(sid * RPT, RPT)],
                    out_hbm.at[pl.ds(cid * NP + sid * RPT, RPT)])


@functools.partial(
    pl.kernel,
    out_type=jax.ShapeDtypeStruct((NC * NP, 16), jnp.float32),
    mesh=_mesh,
    scratch_types=[
        pltpu.VMEM((2, C1), jnp.int32),
        pltpu.VMEM((2, C1), jnp.int32),
        pltpu.VMEM((2, C1, 16), jnp.float32),
        pltpu.SemaphoreType.DMA,
        pltpu.SemaphoreType.DMA,
        pltpu.VMEM_SHARED((NP, 16), jnp.float32),
    ],
    compiler_params=_sc_params,
)
def _sc_agg30(ei_hbm, ta_hbm, tb_hbm, out_hbm,
              src_v, dst_v, rows_v, gsem, isem, acc):
    cid = lax.axis_index("c")
    sid = lax.axis_index("s")
    nchunk = EPT // C1

    @pl.loop(0, ZR1)
    def _(j):
        rows_v[0, j, :] = jnp.zeros((16,), jnp.float32)

    @pl.loop(0, RPT // ZR1)
    def _(j):
        pltpu.sync_copy(rows_v.at[0, pl.ds(0, ZR1), :],
                        acc.at[pl.ds(sid * RPT + j * ZR1, ZR1), :])

    plsc.subcore_barrier()

    nedge = EPT // C1

    def _buf_prep(k, b):
        off = sid * EPT + k * C1
        pltpu.async_copy(ei_hbm.at[0, pl.ds(off, C1)], src_v.at[b], isem)
        pltpu.async_copy(ei_hbm.at[1, pl.ds(off, C1)], dst_v.at[b], isem)
        pltpu.make_async_copy(ei_hbm.at[0, pl.ds(0, C1)],
                              src_v.at[b], isem).wait()
        pltpu.make_async_copy(ei_hbm.at[1, pl.ds(0, C1)],
                              dst_v.at[b], isem).wait()

    H = C1 // 2

    def _gather_start(k, b):
        @pl.when(cid == 0)
        def _():
            pltpu.async_copy(ta_hbm.at[src_v.at[b, pl.ds(0, H)]],
                             rows_v.at[b, pl.ds(0, H), :], gsem)
            pltpu.async_copy(ta_hbm.at[src_v.at[b, pl.ds(H, H)]],
                             rows_v.at[b, pl.ds(H, H), :], gsem)

        @pl.when(cid == 1)
        def _():
            pltpu.async_copy(tb_hbm.at[src_v.at[b, pl.ds(0, H)]],
                             rows_v.at[b, pl.ds(0, H), :], gsem)
            pltpu.async_copy(tb_hbm.at[src_v.at[b, pl.ds(H, H)]],
                             rows_v.at[b, pl.ds(H, H), :], gsem)

    def _gather_wait(b):
        pltpu.make_async_copy(ta_hbm.at[src_v.at[b]], rows_v.at[b],
                              gsem).wait()

    _buf_prep(0, 0)
    _gather_start(0, 0)
    _buf_prep(1, 1)

    @pl.loop(0, nchunk)
    def _(k):
        b = lax.rem(k, 2)
        nb = 1 - b
        _gather_wait(b)

        @pl.when(k + 1 < nchunk)
        def _():
            _gather_start(k + 1, nb)

        pltpu.sync_copy(rows_v.at[b], acc.at[dst_v.at[b]], add=True)

        @pl.when(k + 2 < nchunk)
        def _():
            _buf_prep(k + 2, b)

    plsc.subcore_barrier()
    pltpu.sync_copy(acc.at[pl.ds(sid * RPT, RPT), :],
                    out_hbm.at[pl.ds(cid * NP + sid * RPT, RPT), :])


@functools.partial(
    pl.kernel,
    out_type=jax.ShapeDtypeStruct((NC * NP,), jnp.float32),
    mesh=_mesh,
    scratch_types=[
        pltpu.VMEM((2, C), jnp.int32),
        pltpu.VMEM((2, C), jnp.int32),
        pltpu.VMEM((C,), jnp.float32),
        pltpu.VMEM((NP,), jnp.float32),
        pltpu.VMEM((ZR,), jnp.float32),
        pltpu.SemaphoreType.DMA,
        pltpu.VMEM_SHARED((NP,), jnp.float32),
    ],
    compiler_params=_sc_params,
)
def _sc_agg1(ei_hbm, st_hbm, out_hbm,
             src_v, dst_v, vals_v, table_v, z_v, isem, acc):
    cid = lax.axis_index("c")
    sid = lax.axis_index("s")
    wid = sid * NC + cid
    nchunk = EPW // C
    pltpu.sync_copy(st_hbm.at[0], table_v)
    _fill(z_v, ZR, 0.0)

    @pl.loop(0, RPT // ZR)
    def _(j):
        pltpu.sync_copy(z_v, acc.at[pl.ds(sid * RPT + j * ZR, ZR)])

    plsc.subcore_barrier()

    def _idx(k, b):
        off = wid * EPW + k * C
        pltpu.async_copy(ei_hbm.at[0, pl.ds(off, C)], src_v.at[b], isem)
        pltpu.async_copy(ei_hbm.at[1, pl.ds(off, C)], dst_v.at[b], isem)

    def _idx_wait(b):
        pltpu.make_async_copy(ei_hbm.at[0, pl.ds(0, C)],
                              src_v.at[b], isem).wait()
        pltpu.make_async_copy(ei_hbm.at[1, pl.ds(0, C)],
                              dst_v.at[b], isem).wait()

    _idx(0, 0)

    @pl.loop(0, nchunk)
    def _(k):
        b = lax.rem(k, 2)
        _idx_wait(b)

        @pl.when(k + 1 < nchunk)
        def _():
            _idx(k + 1, 1 - b)

        @pl.loop(0, C // 16)
        def _(j):
            idx = src_v[b, pl.ds(j * 16, 16)]
            vals_v[pl.ds(j * 16, 16)] = plsc.load_gather(table_v, [idx])

        pltpu.sync_copy(vals_v, acc.at[dst_v.at[b]], add=True)

    plsc.subcore_barrier()
    pltpu.sync_copy(acc.at[pl.ds(sid * RPT, RPT)],
                    out_hbm.at[pl.ds(cid * NP + sid * RPT, RPT)])


def _tc1_body(x_ref, w1_ref, p0_ref, p1_ref, ta_ref, tb_ref, d_ref):
    deg = p0_ref[...] + p1_ref[...] + 1.0
    dis = lax.rsqrt(deg)
    h = jnp.dot(x_ref[...], w1_ref[...], preferred_element_type=jnp.float32)
    ht = h * dis[:, None]
    ta_ref[...] = ht[:, :16]
    tb_ref[...] = ht[:, 16:]
    d_ref[...] = dis


def _tc2_body(aa_ref, ab_ref, ta_ref, tb_ref, d_ref, b1a_ref,
              b1b_ref, w2a_ref, w2b_ref, st_ref):
    dd = d_ref[...][:, None]
    h1a = jax.nn.relu(dd * (aa_ref[...] + ta_ref[...]) + b1a_ref[...])
    h1b = jax.nn.relu(dd * (ab_ref[...] + tb_ref[...]) + b1b_ref[...])
    s = (jnp.dot(h1a, w2a_ref[...], preferred_element_type=jnp.float32)
         + jnp.dot(h1b, w2b_ref[...], preferred_element_type=jnp.float32))
    st_ref[...] = jnp.transpose(s, (1, 0)) * d_ref[...][None, :]


def _tc3_body(q0_ref, q1_ref, st_ref, d_ref, b2_ref, o_ref):
    z = (d_ref[...] * (q0_ref[...] + q1_ref[...] + st_ref[...][0])
         + b2_ref[0])
    o_ref[...] = jax.nn.sigmoid(z)


def _rows(width):
    return pl.BlockSpec((BR, width), lambda i: (i, 0))


def _rows2(width):
    return pl.BlockSpec((BR, width), lambda i: (i + GRID, 0))


def _flat():
    return pl.BlockSpec((BR,), lambda i: (i,))


def _flat2():
    return pl.BlockSpec((BR,), lambda i: (i + GRID,))


def _whole(shape):
    return pl.BlockSpec(shape, lambda i: (0,) * len(shape))


_tc1 = pl.pallas_call(
    _tc1_body,
    grid=(GRID,),
    in_specs=[_rows(32), _whole((32, 32)), _flat(), _flat2()],
    out_specs=[_rows(16), _rows(16), _flat()],
    out_shape=[
        jax.ShapeDtypeStruct((NP, 16), jnp.float32),
        jax.ShapeDtypeStruct((NP, 16), jnp.float32),
        jax.ShapeDtypeStruct((NP,), jnp.float32),
    ],
)

_tc2 = pl.pallas_call(
    _tc2_body,
    grid=(GRID,),
    in_specs=[_rows(16), _rows2(16), _rows(16), _rows(16),
              _flat(), _whole((1, 16)), _whole((1, 16)),
              _whole((16, 1)), _whole((16, 1))],
    out_specs=pl.BlockSpec((1, BR), lambda i: (0, i)),
    out_shape=jax.ShapeDtypeStruct((1, NP), jnp.float32),
)

_tc3 = pl.pallas_call(
    _tc3_body,
    grid=(GRID,),
    in_specs=[_flat(), _flat2(), pl.BlockSpec((1, BR), lambda i: (0, i)),
              _flat(), _whole((1,))],
    out_specs=_flat(),
    out_shape=jax.ShapeDtypeStruct((NP,), jnp.float32),
)


def kernel(x, edge_index, W1, b1, W2, b2):
    xp = jnp.pad(x, ((0, NP - N), (0, 0)))
    w1p = jnp.pad(W1, ((0, 0), (0, 2)))
    b1p = jnp.pad(b1, (0, 2)).reshape(1, 32)
    w2p = jnp.pad(W2[:, 0], (0, 2)).reshape(1, 32)

    degp = _sc_deg(edge_index)

    ta, tb, d = _tc1(xp, w1p, degp, degp)

    accf = _sc_agg30(edge_index, ta, tb)

    st = _tc2(accf, accf, ta, tb, d, b1p[:, :16], b1p[:, 16:],
              w2p[0, :16].reshape(16, 1), w2p[0, 16:].reshape(16, 1))

    acc2 = _sc_agg1(edge_index, st)

    out = _tc3(acc2, acc2, st, d, b2)
    return out[:N, None]
